# baseline (speedup 1.0000x reference)
import jax
import jax.numpy as jnp
from jax import lax
from jax.experimental import pallas as pl
from jax.experimental.pallas import tpu as pltpu
from jax.experimental.pallas import tpu_sc as plsc

_BATCH = 4096
_NN = 16384
_SY = 8192
_NW = 32
_ROWS_PER_W = _BATCH // _NW
_RPC = 2
_G = _ROWS_PER_W // _RPC


def _sc_body(act, comb, out, comb_v, in0, in1, out0, out1, si0, si1, so0, so1):
    c = lax.axis_index("c")
    s = lax.axis_index("s")
    wid = s * 2 + c
    rowbase = wid * _ROWS_PER_W

    pltpu.sync_copy(comb, comb_v)

    ins = (in0, in1)
    outs = (out0, out1)
    sis = (si0, si1)
    sos = (so0, so1)

    def in_copies(g, b):
        row = rowbase + _RPC * g
        return (pltpu.make_async_copy(act.at[pl.ds(row, _RPC)], ins[b], sis[b]),)

    def out_copies(g, b):
        row = rowbase + _RPC * g
        return (pltpu.make_async_copy(outs[b], out.at[pl.ds(row, _RPC)], sos[b]),)

    def start(copies):
        for cp in copies:
            cp.start()

    def wait(copies):
        for cp in copies:
            cp.wait()

    row0 = jnp.zeros((16,), jnp.int32)
    row1 = jnp.ones((16,), jnp.int32)

    def compute(b):
        inb = ins[b]
        outb = outs[b]

        @plsc.parallel_loop(0, _SY // 16, unroll=16)
        def _(j):
            cv = comb_v[pl.ds(j * 16, 16)]
            il = cv & 0xFFFF
            ir = cv >> 16
            a0 = plsc.load_gather(inb, [row0, il])
            b0 = plsc.load_gather(inb, [row0, ir])
            outb[0, pl.ds(j * 16, 16)] = a0 * b0
            a1 = plsc.load_gather(inb, [row1, il])
            b1 = plsc.load_gather(inb, [row1, ir])
            outb[1, pl.ds(j * 16, 16)] = a1 * b1

    start(in_copies(0, 0))
    start(in_copies(1, 1))

    for b in range(2):
        wait(in_copies(b, b))
        compute(b)
        start(out_copies(b, b))
        start(in_copies(b + 2, b))

    def outer(gg, carry):
        for b in range(2):
            g = 2 * gg + b
            wait(in_copies(g, b))
            wait(out_copies(g - 2, b))
            compute(b)
            start(out_copies(g, b))
            start(in_copies(g + 2, b))
        return carry

    lax.fori_loop(1, _G // 2 - 1, outer, None)

    for b in range(2):
        g = _G - 2 + b
        wait(in_copies(g, b))
        wait(out_copies(g - 2, b))
        compute(b)
        start(out_copies(g, b))
    for b in range(2):
        wait(out_copies(_G - 2 + b, b))


def kernel(post_activations, left_indices, right_indices):
    li = left_indices.astype(jnp.int32)
    ri = right_indices.astype(jnp.int32)
    comb = li | (ri << 16)

    mesh = plsc.VectorSubcoreMesh(core_axis_name="c", subcore_axis_name="s")
    f = pl.kernel(
        _sc_body,
        out_type=jax.ShapeDtypeStruct((_BATCH, _SY), jnp.float32),
        mesh=mesh,
        scratch_types=[
            pltpu.VMEM((_SY,), jnp.int32),
            pltpu.VMEM((_RPC, _NN), jnp.float32),
            pltpu.VMEM((_RPC, _NN), jnp.float32),
            pltpu.VMEM((_RPC, _SY), jnp.float32),
            pltpu.VMEM((_RPC, _SY), jnp.float32),
            pltpu.SemaphoreType.DMA,
            pltpu.SemaphoreType.DMA,
            pltpu.SemaphoreType.DMA,
            pltpu.SemaphoreType.DMA,
        ],
        compiler_params=pltpu.CompilerParams(needs_layout_passes=False),
    )
    return f(post_activations, comb)


# speedup vs baseline: 1.8483x; 1.8483x over previous
import jax
import jax.numpy as jnp
from jax import lax
from jax.experimental import pallas as pl
from jax.experimental.pallas import tpu as pltpu
from jax.experimental.pallas import tpu_sc as plsc

_BATCH = 4096
_NN = 16384
_SY = 8192
_NW = 32
_ROWS_PER_W = _BATCH // _NW
_RPC = 2
_G = _ROWS_PER_W // _RPC


def _sc_body(act, comb, out, comb_v, in0, in1, out0, out1, si0, si1, so0, so1):
    c = lax.axis_index("c")
    s = lax.axis_index("s")
    wid = s * 2 + c
    rowbase = wid * _ROWS_PER_W

    pltpu.sync_copy(comb, comb_v)

    ins = (in0, in1)
    outs = (out0, out1)
    sis = (si0, si1)
    sos = (so0, so1)

    def in_copies(g, b):
        row = rowbase + _RPC * g
        return (
            pltpu.make_async_copy(act.at[row], ins[b].at[pl.ds(0, _NN)], sis[b]),
            pltpu.make_async_copy(act.at[row + 1], ins[b].at[pl.ds(_NN, _NN)], sis[b]),
        )

    def out_copies(g, b):
        row = rowbase + _RPC * g
        return (
            pltpu.make_async_copy(outs[b].at[pl.ds(0, _SY)], out.at[row], sos[b]),
            pltpu.make_async_copy(outs[b].at[pl.ds(_SY, _SY)], out.at[row + 1], sos[b]),
        )

    def start(copies):
        for cp in copies:
            cp.start()

    def wait(copies):
        for cp in copies:
            cp.wait()

    def compute(b):
        inb = ins[b]
        outb = outs[b]

        inb1 = inb.at[pl.ds(_NN, _NN)]

        @plsc.parallel_loop(0, _SY // 16, unroll=8)
        def _(j):
            cv = comb_v[pl.ds(j * 16, 16)]
            il = cv & 0xFFFF
            ir = cv >> 16
            a0 = plsc.load_gather(inb, [il])
            b0 = plsc.load_gather(inb, [ir])
            outb[pl.ds(j * 16, 16)] = a0 * b0
            a1 = plsc.load_gather(inb1, [il])
            b1 = plsc.load_gather(inb1, [ir])
            outb[pl.ds(_SY + j * 16, 16)] = a1 * b1

    start(in_copies(0, 0))
    start(in_copies(1, 1))

    for b in range(2):
        wait(in_copies(b, b))
        compute(b)
        start(out_copies(b, b))
        start(in_copies(b + 2, b))

    def outer(gg, carry):
        for b in range(2):
            g = 2 * gg + b
            wait(in_copies(g, b))
            wait(out_copies(g - 2, b))
            compute(b)
            start(out_copies(g, b))
            start(in_copies(g + 2, b))
        return carry

    lax.fori_loop(1, _G // 2 - 1, outer, None)

    for b in range(2):
        g = _G - 2 + b
        wait(in_copies(g, b))
        wait(out_copies(g - 2, b))
        compute(b)
        start(out_copies(g, b))
    for b in range(2):
        wait(out_copies(_G - 2 + b, b))


def kernel(post_activations, left_indices, right_indices):
    li = left_indices.astype(jnp.int32)
    ri = right_indices.astype(jnp.int32)
    comb = li | (ri << 16)

    mesh = plsc.VectorSubcoreMesh(core_axis_name="c", subcore_axis_name="s")
    f = pl.kernel(
        _sc_body,
        out_type=jax.ShapeDtypeStruct((_BATCH, _SY), jnp.float32),
        mesh=mesh,
        scratch_types=[
            pltpu.VMEM((_SY,), jnp.int32),
            pltpu.VMEM((_RPC * _NN,), jnp.float32),
            pltpu.VMEM((_RPC * _NN,), jnp.float32),
            pltpu.VMEM((_RPC * _SY,), jnp.float32),
            pltpu.VMEM((_RPC * _SY,), jnp.float32),
            pltpu.SemaphoreType.DMA,
            pltpu.SemaphoreType.DMA,
            pltpu.SemaphoreType.DMA,
            pltpu.SemaphoreType.DMA,
        ],
        compiler_params=pltpu.CompilerParams(needs_layout_passes=False),
    )
    return f(post_activations, comb)
